# baseline (speedup 1.0000x reference)
import functools

import jax
import jax.numpy as jnp
import numpy as np
from jax.experimental import pallas as pl

_NUM_TABLES = 4
_NUM_BITS = 8
_TOP_K = 100
_BLK = 2048
_NEG = -3.0e38


def _lin_kernel(hid_ref, w1_ref, b1_ref, x_ref):
    x_ref[...] = jnp.dot(hid_ref[...], w1_ref[...],
                         preferred_element_type=jnp.float32) + b1_ref[...]


def _score_kernel(n_items, q_ref, items_ref, p_ref, pt_ref, mt_ref, out_ref):
    j = pl.program_id(0)
    items = items_ref[...]
    q = q_ref[...]
    projq = jnp.dot(q, p_ref[...], preferred_element_type=jnp.float32)
    bitsq = (projq > 0).astype(jnp.float32)
    qc = jax.lax.dot_general(bitsq, mt_ref[...], (((1,), (1,)), ((), ())),
                             preferred_element_type=jnp.float32
                             ).astype(jnp.int32)
    projt = jax.lax.dot_general(pt_ref[...], items, (((1,), (1,)), ((), ())),
                                preferred_element_type=jnp.float32)
    bitst = (projt > 0).astype(jnp.float32)
    codest = jnp.dot(mt_ref[...], bitst,
                     preferred_element_type=jnp.float32).astype(jnp.int32)
    mask = qc[:, 0:1] == codest[0:1, :]
    for t in range(1, _NUM_TABLES):
        mask = mask | (qc[:, t:t + 1] == codest[t:t + 1, :])
    scores = jax.lax.dot_general(q, items, (((1,), (1,)), ((), ())),
                                 preferred_element_type=jnp.float32)
    boosted = jnp.where(mask, scores + 1e6, scores)
    col = j * _BLK + jax.lax.broadcasted_iota(jnp.int32, boosted.shape, 1)
    out_ref[...] = jnp.where(col < n_items, boosted, _NEG)


def kernel(hidden_state, item_embeddings, W1, b1, gamma, beta, lsh_proj):
    B, _ = hidden_state.shape
    N, D = item_embeddings.shape
    T, _, H = lsh_proj.shape
    TH = T * H

    x = pl.pallas_call(
        _lin_kernel,
        out_shape=jax.ShapeDtypeStruct((B, D), jnp.float32),
    )(hidden_state, W1, b1.reshape(1, D))

    mu = jnp.mean(x, axis=-1, keepdims=True)
    var = jnp.var(x, axis=-1, keepdims=True)
    q = jnp.tanh((x - mu) / jnp.sqrt(var + 1e-5) * gamma + beta)

    p_mat = jnp.transpose(lsh_proj, (1, 0, 2)).reshape(D, TH)
    m_np = np.zeros((T, TH), dtype=np.float32)
    for t in range(T):
        for h in range(H):
            m_np[t, t * H + h] = float(2 ** h)
    mt_mat = jnp.asarray(m_np)

    npad = ((N + _BLK - 1) // _BLK) * _BLK
    items_p = jnp.concatenate(
        [item_embeddings, jnp.zeros((npad - N, D), jnp.float32)], axis=0)
    nblk = npad // _BLK

    boosted = pl.pallas_call(
        functools.partial(_score_kernel, N),
        grid=(nblk,),
        in_specs=[
            pl.BlockSpec((B, D), lambda j: (0, 0)),
            pl.BlockSpec((_BLK, D), lambda j: (j, 0)),
            pl.BlockSpec((D, TH), lambda j: (0, 0)),
            pl.BlockSpec((TH, D), lambda j: (0, 0)),
            pl.BlockSpec((T, TH), lambda j: (0, 0)),
        ],
        out_specs=pl.BlockSpec((B, _BLK), lambda j: (0, j)),
        out_shape=jax.ShapeDtypeStruct((B, npad), jnp.float32),
    )(q, items_p, p_mat, p_mat.T, mt_mat)

    _, candidate_ids = jax.lax.top_k(boosted, _TOP_K)
    candidate_embeddings = jnp.take(item_embeddings, candidate_ids, axis=0)
    return (q, candidate_ids, candidate_embeddings)


# speedup vs baseline: 2.5662x; 2.5662x over previous
import functools

import jax
import jax.numpy as jnp
import numpy as np
from jax.experimental import pallas as pl

_NUM_TABLES = 4
_NUM_BITS = 8
_TOP_K = 100
_BLK = 2048
_SUB = 128
_NEG = -3.0e38


def _lin_kernel(hid_ref, w1_ref, b1_ref, x_ref):
    x_ref[...] = jnp.dot(hid_ref[...], w1_ref[...],
                         preferred_element_type=jnp.float32) + b1_ref[...]


def _score_kernel(n_items, q_ref, items_ref, p_ref, pt_ref, mt_ref,
                  out_ref, max_ref):
    j = pl.program_id(0)
    items = items_ref[...]
    q = q_ref[...]
    projq = jnp.dot(q, p_ref[...], preferred_element_type=jnp.float32)
    bitsq = (projq > 0).astype(jnp.float32)
    qc = jax.lax.dot_general(bitsq, mt_ref[...], (((1,), (1,)), ((), ())),
                             preferred_element_type=jnp.float32
                             ).astype(jnp.int32)
    projt = jax.lax.dot_general(pt_ref[...], items, (((1,), (1,)), ((), ())),
                                preferred_element_type=jnp.float32)
    bitst = (projt > 0).astype(jnp.float32)
    codest = jnp.dot(mt_ref[...], bitst,
                     preferred_element_type=jnp.float32).astype(jnp.int32)
    mask = qc[:, 0:1] == codest[0:1, :]
    for t in range(1, _NUM_TABLES):
        mask = mask | (qc[:, t:t + 1] == codest[t:t + 1, :])
    scores = jax.lax.dot_general(q, items, (((1,), (1,)), ((), ())),
                                 preferred_element_type=jnp.float32)
    boosted = jnp.where(mask, scores + 1e6, scores)
    col = j * _BLK + jax.lax.broadcasted_iota(jnp.int32, boosted.shape, 1)
    boosted = jnp.where(col < n_items, boosted, _NEG)
    out_ref[...] = boosted
    b = boosted.shape[0]
    max_ref[...] = jnp.max(boosted.reshape(b, _BLK // _SUB, _SUB),
                           axis=-1).reshape(1, b, _BLK // _SUB)


def kernel(hidden_state, item_embeddings, W1, b1, gamma, beta, lsh_proj):
    B, _ = hidden_state.shape
    N, D = item_embeddings.shape
    T, _, H = lsh_proj.shape
    TH = T * H

    x = pl.pallas_call(
        _lin_kernel,
        out_shape=jax.ShapeDtypeStruct((B, D), jnp.float32),
    )(hidden_state, W1, b1.reshape(1, D))

    mu = jnp.mean(x, axis=-1, keepdims=True)
    var = jnp.var(x, axis=-1, keepdims=True)
    q = jnp.tanh((x - mu) / jnp.sqrt(var + 1e-5) * gamma + beta)

    p_mat = jnp.transpose(lsh_proj, (1, 0, 2)).reshape(D, TH)
    m_np = np.zeros((T, TH), dtype=np.float32)
    for t in range(T):
        for h in range(H):
            m_np[t, t * H + h] = float(2 ** h)
    mt_mat = jnp.asarray(m_np)

    npad = ((N + _BLK - 1) // _BLK) * _BLK
    items_p = jnp.concatenate(
        [item_embeddings, jnp.zeros((npad - N, D), jnp.float32)], axis=0)
    nblk = npad // _BLK
    nsub = npad // _SUB

    boosted, submax = pl.pallas_call(
        functools.partial(_score_kernel, N),
        grid=(nblk,),
        in_specs=[
            pl.BlockSpec((B, D), lambda j: (0, 0)),
            pl.BlockSpec((_BLK, D), lambda j: (j, 0)),
            pl.BlockSpec((D, TH), lambda j: (0, 0)),
            pl.BlockSpec((TH, D), lambda j: (0, 0)),
            pl.BlockSpec((T, TH), lambda j: (0, 0)),
        ],
        out_specs=[
            pl.BlockSpec((B, _BLK), lambda j: (0, j)),
            pl.BlockSpec((1, B, _BLK // _SUB), lambda j: (j, 0, 0)),
        ],
        out_shape=[
            jax.ShapeDtypeStruct((B, npad), jnp.float32),
            jax.ShapeDtypeStruct((nblk, B, _BLK // _SUB), jnp.float32),
        ],
    )(q, items_p, p_mat, p_mat.T, mt_mat)

    submax = jnp.transpose(submax, (1, 0, 2)).reshape(B, nsub)
    _, blk_ids = jax.lax.top_k(submax, _TOP_K)
    blocks = boosted.reshape(B, nsub, _SUB)
    gathered = jnp.take_along_axis(blocks, blk_ids[:, :, None], axis=1)
    flat = gathered.reshape(B, _TOP_K * _SUB)
    orig = (blk_ids[:, :, None] * _SUB
            + jnp.arange(_SUB, dtype=jnp.int32)[None, None, :]
            ).reshape(B, _TOP_K * _SUB)

    vals, pos = jax.lax.top_k(flat, _TOP_K)
    tau = vals[:, -1:]
    g = jnp.sum((vals > tau).astype(jnp.int32), axis=1, keepdims=True)

    gt_ids = jnp.take_along_axis(orig, pos, axis=1)
    gt_valid = vals > tau
    imin = jnp.int32(-2147483647)
    eqkey = jnp.where(flat == tau, -orig, imin)
    eqk, _ = jax.lax.top_k(eqkey, _TOP_K)
    eq_ids = -eqk
    kio = jnp.arange(_TOP_K, dtype=jnp.int32)[None, :]
    eq_valid = (kio < (_TOP_K - g)) & (eqk != imin)

    pool_ids = jnp.concatenate([jnp.where(gt_valid, gt_ids, 0),
                                jnp.where(eq_valid, eq_ids, 0)], axis=1)
    pool_vals = jnp.concatenate(
        [jnp.where(gt_valid, vals, -jnp.inf),
         jnp.where(eq_valid, jnp.broadcast_to(tau, eq_ids.shape), -jnp.inf)],
        axis=1)
    sneg, sids = jax.lax.sort((-pool_vals, pool_ids), dimension=1, num_keys=2)
    candidate_ids = sids[:, :_TOP_K]
    candidate_embeddings = jnp.take(item_embeddings, candidate_ids, axis=0)
    return (q, candidate_ids, candidate_embeddings)


# speedup vs baseline: 5.8482x; 2.2790x over previous
import functools

import jax
import jax.numpy as jnp
import numpy as np
from jax.experimental import pallas as pl

_NUM_TABLES = 4
_NUM_BITS = 8
_TOP_K = 100
_BLK = 2048
_SUB = 128
_NEG = -3.0e38


def _lin_kernel(hid_ref, w1_ref, b1_ref, x_ref):
    x_ref[...] = jnp.dot(hid_ref[...], w1_ref[...],
                         preferred_element_type=jnp.float32) + b1_ref[...]


def _score_kernel(n_items, q_ref, items_ref, p_ref, pt_ref, mt_ref,
                  out_ref, max_ref):
    j = pl.program_id(0)
    items = items_ref[...]
    q = q_ref[...]
    projq = jnp.dot(q, p_ref[...], preferred_element_type=jnp.float32)
    bitsq = (projq > 0).astype(jnp.float32)
    qc = jax.lax.dot_general(bitsq, mt_ref[...], (((1,), (1,)), ((), ())),
                             preferred_element_type=jnp.float32
                             ).astype(jnp.int32)
    projt = jax.lax.dot_general(pt_ref[...], items, (((1,), (1,)), ((), ())),
                                preferred_element_type=jnp.float32)
    bitst = (projt > 0).astype(jnp.float32)
    codest = jnp.dot(mt_ref[...], bitst,
                     preferred_element_type=jnp.float32).astype(jnp.int32)
    mask = qc[:, 0:1] == codest[0:1, :]
    for t in range(1, _NUM_TABLES):
        mask = mask | (qc[:, t:t + 1] == codest[t:t + 1, :])
    scores = jax.lax.dot_general(q, items, (((1,), (1,)), ((), ())),
                                 preferred_element_type=jnp.float32)
    boosted = jnp.where(mask, scores + 1e6, scores)
    col = j * _BLK + jax.lax.broadcasted_iota(jnp.int32, boosted.shape, 1)
    boosted = jnp.where(col < n_items, boosted, _NEG)
    out_ref[...] = boosted
    b = boosted.shape[0]
    max_ref[...] = jnp.max(boosted.reshape(b, _BLK // _SUB, _SUB),
                           axis=-1).reshape(1, b, _BLK // _SUB)


def kernel(hidden_state, item_embeddings, W1, b1, gamma, beta, lsh_proj):
    B, _ = hidden_state.shape
    N, D = item_embeddings.shape
    T, _, H = lsh_proj.shape
    TH = T * H

    x = pl.pallas_call(
        _lin_kernel,
        out_shape=jax.ShapeDtypeStruct((B, D), jnp.float32),
    )(hidden_state, W1, b1.reshape(1, D))

    mu = jnp.mean(x, axis=-1, keepdims=True)
    var = jnp.var(x, axis=-1, keepdims=True)
    q = jnp.tanh((x - mu) / jnp.sqrt(var + 1e-5) * gamma + beta)

    p_mat = jnp.transpose(lsh_proj, (1, 0, 2)).reshape(D, TH)
    m_np = np.zeros((T, TH), dtype=np.float32)
    for t in range(T):
        for h in range(H):
            m_np[t, t * H + h] = float(2 ** h)
    mt_mat = jnp.asarray(m_np)

    npad = ((N + _BLK - 1) // _BLK) * _BLK
    items_p = jnp.concatenate(
        [item_embeddings, jnp.zeros((npad - N, D), jnp.float32)], axis=0)
    nblk = npad // _BLK
    nsub = npad // _SUB

    boosted, submax = pl.pallas_call(
        functools.partial(_score_kernel, N),
        grid=(nblk,),
        in_specs=[
            pl.BlockSpec((B, D), lambda j: (0, 0)),
            pl.BlockSpec((_BLK, D), lambda j: (j, 0)),
            pl.BlockSpec((D, TH), lambda j: (0, 0)),
            pl.BlockSpec((TH, D), lambda j: (0, 0)),
            pl.BlockSpec((T, TH), lambda j: (0, 0)),
        ],
        out_specs=[
            pl.BlockSpec((B, _BLK), lambda j: (0, j)),
            pl.BlockSpec((1, B, _BLK // _SUB), lambda j: (j, 0, 0)),
        ],
        out_shape=[
            jax.ShapeDtypeStruct((B, npad), jnp.float32),
            jax.ShapeDtypeStruct((nblk, B, _BLK // _SUB), jnp.float32),
        ],
    )(q, items_p, p_mat, p_mat.T, mt_mat)

    submax = jnp.transpose(submax, (1, 0, 2)).reshape(B, nsub)
    _, blk_ids = jax.lax.top_k(submax, _TOP_K)
    blk_ids = jnp.sort(blk_ids, axis=1)
    blocks = boosted.reshape(B, nsub, _SUB)
    gathered = jnp.take_along_axis(blocks, blk_ids[:, :, None], axis=1)
    flat = gathered.reshape(B, _TOP_K * _SUB)
    orig = (blk_ids[:, :, None] * _SUB
            + jnp.arange(_SUB, dtype=jnp.int32)[None, None, :]
            ).reshape(B, _TOP_K * _SUB)
    _, pos = jax.lax.top_k(flat, _TOP_K)
    candidate_ids = jnp.take_along_axis(orig, pos, axis=1)
    candidate_embeddings = jnp.take(item_embeddings, candidate_ids, axis=0)
    return (q, candidate_ids, candidate_embeddings)


# speedup vs baseline: 6.8371x; 1.1691x over previous
import functools

import jax
import jax.numpy as jnp
import numpy as np
from jax.experimental import pallas as pl

_NUM_TABLES = 4
_NUM_BITS = 8
_TOP_K = 100
_BLK = 2048
_SUB = 64
_NEG = -3.0e38


def _lin_kernel(hid_ref, w1_ref, b1_ref, x_ref):
    x_ref[...] = jnp.dot(hid_ref[...], w1_ref[...],
                         preferred_element_type=jnp.float32) + b1_ref[...]


def _score_kernel(n_items, q_ref, items_ref, p_ref, pt_ref, mt_ref,
                  out_ref, max_ref):
    j = pl.program_id(0)
    items = items_ref[...]
    q = q_ref[...]
    projq = jnp.dot(q, p_ref[...], preferred_element_type=jnp.float32)
    bitsq = (projq > 0).astype(jnp.float32)
    qc = jax.lax.dot_general(bitsq, mt_ref[...], (((1,), (1,)), ((), ())),
                             preferred_element_type=jnp.float32
                             ).astype(jnp.int32)
    projt = jax.lax.dot_general(pt_ref[...], items, (((1,), (1,)), ((), ())),
                                preferred_element_type=jnp.float32)
    bitst = (projt > 0).astype(jnp.float32)
    codest = jnp.dot(mt_ref[...], bitst,
                     preferred_element_type=jnp.float32).astype(jnp.int32)
    mask = qc[:, 0:1] == codest[0:1, :]
    for t in range(1, _NUM_TABLES):
        mask = mask | (qc[:, t:t + 1] == codest[t:t + 1, :])
    scores = jax.lax.dot_general(q, items, (((1,), (1,)), ((), ())),
                                 preferred_element_type=jnp.float32)
    boosted = jnp.where(mask, scores + 1e6, scores)
    col = j * _BLK + jax.lax.broadcasted_iota(jnp.int32, boosted.shape, 1)
    boosted = jnp.where(col < n_items, boosted, _NEG)
    out_ref[...] = boosted
    b = boosted.shape[0]
    max_ref[...] = jnp.max(boosted.reshape(b, _BLK // _SUB, _SUB),
                           axis=-1).reshape(1, b, _BLK // _SUB)


def kernel(hidden_state, item_embeddings, W1, b1, gamma, beta, lsh_proj):
    B, _ = hidden_state.shape
    N, D = item_embeddings.shape
    T, _, H = lsh_proj.shape
    TH = T * H

    x = pl.pallas_call(
        _lin_kernel,
        out_shape=jax.ShapeDtypeStruct((B, D), jnp.float32),
    )(hidden_state, W1, b1.reshape(1, D))

    mu = jnp.mean(x, axis=-1, keepdims=True)
    var = jnp.var(x, axis=-1, keepdims=True)
    q = jnp.tanh((x - mu) / jnp.sqrt(var + 1e-5) * gamma + beta)

    p_mat = jnp.transpose(lsh_proj, (1, 0, 2)).reshape(D, TH)
    m_np = np.zeros((T, TH), dtype=np.float32)
    for t in range(T):
        for h in range(H):
            m_np[t, t * H + h] = float(2 ** h)
    mt_mat = jnp.asarray(m_np)

    npad = ((N + _BLK - 1) // _BLK) * _BLK
    items_p = jnp.concatenate(
        [item_embeddings, jnp.zeros((npad - N, D), jnp.float32)], axis=0)
    nblk = npad // _BLK
    nsub = npad // _SUB

    boosted, submax = pl.pallas_call(
        functools.partial(_score_kernel, N),
        grid=(nblk,),
        in_specs=[
            pl.BlockSpec((B, D), lambda j: (0, 0)),
            pl.BlockSpec((_BLK, D), lambda j: (j, 0)),
            pl.BlockSpec((D, TH), lambda j: (0, 0)),
            pl.BlockSpec((TH, D), lambda j: (0, 0)),
            pl.BlockSpec((T, TH), lambda j: (0, 0)),
        ],
        out_specs=[
            pl.BlockSpec((B, _BLK), lambda j: (0, j)),
            pl.BlockSpec((1, B, _BLK // _SUB), lambda j: (j, 0, 0)),
        ],
        out_shape=[
            jax.ShapeDtypeStruct((B, npad), jnp.float32),
            jax.ShapeDtypeStruct((nblk, B, _BLK // _SUB), jnp.float32),
        ],
    )(q, items_p, p_mat, p_mat.T, mt_mat)

    submax = jnp.transpose(submax, (1, 0, 2)).reshape(B, nsub)
    _, blk_ids = jax.lax.top_k(submax, _TOP_K)
    blk_ids = jnp.sort(blk_ids, axis=1)
    blocks = boosted.reshape(B, nsub, _SUB)
    gathered = jnp.take_along_axis(blocks, blk_ids[:, :, None], axis=1)
    flat = gathered.reshape(B, _TOP_K * _SUB)
    orig = (blk_ids[:, :, None] * _SUB
            + jnp.arange(_SUB, dtype=jnp.int32)[None, None, :]
            ).reshape(B, _TOP_K * _SUB)
    _, pos = jax.lax.top_k(flat, _TOP_K)
    candidate_ids = jnp.take_along_axis(orig, pos, axis=1)
    candidate_embeddings = jnp.take(item_embeddings, candidate_ids, axis=0)
    return (q, candidate_ids, candidate_embeddings)


# speedup vs baseline: 7.2477x; 1.0601x over previous
import functools

import jax
import jax.numpy as jnp
import numpy as np
from jax.experimental import pallas as pl

_NUM_TABLES = 4
_NUM_BITS = 8
_TOP_K = 100
_BLK = 2048
_SUB = 32
_NEG = -3.0e38


def _lin_kernel(hid_ref, w1_ref, b1_ref, x_ref):
    x_ref[...] = jnp.dot(hid_ref[...], w1_ref[...],
                         preferred_element_type=jnp.float32) + b1_ref[...]


def _score_kernel(n_items, q_ref, items_ref, p_ref, pt_ref, mt_ref,
                  out_ref, max_ref):
    j = pl.program_id(0)
    items = items_ref[...]
    q = q_ref[...]
    projq = jnp.dot(q, p_ref[...], preferred_element_type=jnp.float32)
    bitsq = (projq > 0).astype(jnp.float32)
    qc = jax.lax.dot_general(bitsq, mt_ref[...], (((1,), (1,)), ((), ())),
                             preferred_element_type=jnp.float32
                             ).astype(jnp.int32)
    projt = jax.lax.dot_general(pt_ref[...], items, (((1,), (1,)), ((), ())),
                                preferred_element_type=jnp.float32)
    bitst = (projt > 0).astype(jnp.float32)
    codest = jnp.dot(mt_ref[...], bitst,
                     preferred_element_type=jnp.float32).astype(jnp.int32)
    mask = qc[:, 0:1] == codest[0:1, :]
    for t in range(1, _NUM_TABLES):
        mask = mask | (qc[:, t:t + 1] == codest[t:t + 1, :])
    scores = jax.lax.dot_general(q, items, (((1,), (1,)), ((), ())),
                                 preferred_element_type=jnp.float32)
    boosted = jnp.where(mask, scores + 1e6, scores)
    col = j * _BLK + jax.lax.broadcasted_iota(jnp.int32, boosted.shape, 1)
    boosted = jnp.where(col < n_items, boosted, _NEG)
    out_ref[...] = boosted
    b = boosted.shape[0]
    max_ref[...] = jnp.max(boosted.reshape(b, _BLK // _SUB, _SUB),
                           axis=-1).reshape(1, b, _BLK // _SUB)


def kernel(hidden_state, item_embeddings, W1, b1, gamma, beta, lsh_proj):
    B, _ = hidden_state.shape
    N, D = item_embeddings.shape
    T, _, H = lsh_proj.shape
    TH = T * H

    x = pl.pallas_call(
        _lin_kernel,
        out_shape=jax.ShapeDtypeStruct((B, D), jnp.float32),
    )(hidden_state, W1, b1.reshape(1, D))

    mu = jnp.mean(x, axis=-1, keepdims=True)
    var = jnp.var(x, axis=-1, keepdims=True)
    q = jnp.tanh((x - mu) / jnp.sqrt(var + 1e-5) * gamma + beta)

    p_mat = jnp.transpose(lsh_proj, (1, 0, 2)).reshape(D, TH)
    m_np = np.zeros((T, TH), dtype=np.float32)
    for t in range(T):
        for h in range(H):
            m_np[t, t * H + h] = float(2 ** h)
    mt_mat = jnp.asarray(m_np)

    npad = ((N + _BLK - 1) // _BLK) * _BLK
    items_p = jnp.concatenate(
        [item_embeddings, jnp.zeros((npad - N, D), jnp.float32)], axis=0)
    nblk = npad // _BLK
    nsub = npad // _SUB

    boosted, submax = pl.pallas_call(
        functools.partial(_score_kernel, N),
        grid=(nblk,),
        in_specs=[
            pl.BlockSpec((B, D), lambda j: (0, 0)),
            pl.BlockSpec((_BLK, D), lambda j: (j, 0)),
            pl.BlockSpec((D, TH), lambda j: (0, 0)),
            pl.BlockSpec((TH, D), lambda j: (0, 0)),
            pl.BlockSpec((T, TH), lambda j: (0, 0)),
        ],
        out_specs=[
            pl.BlockSpec((B, _BLK), lambda j: (0, j)),
            pl.BlockSpec((1, B, _BLK // _SUB), lambda j: (j, 0, 0)),
        ],
        out_shape=[
            jax.ShapeDtypeStruct((B, npad), jnp.float32),
            jax.ShapeDtypeStruct((nblk, B, _BLK // _SUB), jnp.float32),
        ],
    )(q, items_p, p_mat, p_mat.T, mt_mat)

    submax = jnp.transpose(submax, (1, 0, 2)).reshape(B, nsub)
    _, blk_ids = jax.lax.top_k(submax, _TOP_K)
    blk_ids = jnp.sort(blk_ids, axis=1)
    blocks = boosted.reshape(B, nsub, _SUB)
    gathered = jnp.take_along_axis(blocks, blk_ids[:, :, None], axis=1)
    flat = gathered.reshape(B, _TOP_K * _SUB)
    orig = (blk_ids[:, :, None] * _SUB
            + jnp.arange(_SUB, dtype=jnp.int32)[None, None, :]
            ).reshape(B, _TOP_K * _SUB)
    _, pos = jax.lax.top_k(flat, _TOP_K)
    candidate_ids = jnp.take_along_axis(orig, pos, axis=1)
    candidate_embeddings = jnp.take(item_embeddings, candidate_ids, axis=0)
    return (q, candidate_ids, candidate_embeddings)
